# baseline (speedup 1.0000x reference)
import functools
import jax
import jax.numpy as jnp
from jax import lax
from jax.experimental import pallas as pl
from jax.experimental.pallas import tpu as pltpu
from jax.experimental.pallas import tpu_sc as plsc

NUM_ROWS = 1000
D = 128
NSEQ = 4096
SEQ_LEN = 50
NC, NS = 2, 16
NW = NC * NS
S_PER_W = NSEQ // NW


def _make_kernel():
  mesh = plsc.VectorSubcoreMesh(core_axis_name="c", subcore_axis_name="s")

  @functools.partial(
      pl.kernel,
      out_type=jax.ShapeDtypeStruct((SEQ_LEN, NSEQ, D), jnp.float32),
      mesh=mesh,
      scratch_types=[
          pltpu.VMEM((SEQ_LEN, S_PER_W), jnp.int32),
          pltpu.VMEM((4, S_PER_W, D), jnp.float32),
          pltpu.VMEM_SHARED((NUM_ROWS, D), jnp.float32),
          pltpu.SemaphoreType.DMA,
          pltpu.SemaphoreType.DMA,
      ],
  )
  def gather_kernel(idx_hbm, table_hbm, out_hbm, idx_v, rows_v, table_sh,
                    sem_g, sem_s):
    sid = lax.axis_index("s")
    wid = sid * NC + lax.axis_index("c")
    base = wid * S_PER_W
    pltpu.sync_copy(idx_hbm.at[:, pl.ds(base, S_PER_W)], idx_v)

    @pl.when(sid == 0)
    def _stage_table():
      pltpu.sync_copy(table_hbm, table_sh)

    plsc.subcore_barrier()

    pltpu.async_copy(table_sh.at[idx_v.at[0]], rows_v.at[0], sem_g)
    pltpu.async_copy(table_sh.at[idx_v.at[1]], rows_v.at[1], sem_g)

    def _drain_one_scatter():
      pltpu.make_async_copy(
          rows_v.at[0], out_hbm.at[0, pl.ds(base, S_PER_W)], sem_s).wait()

    def body(j, carry):
      b = lax.rem(j, 4)
      nb = lax.rem(j + 2, 4)

      @pl.when(j >= 2)
      def _wait_scatter_jm2():
        _drain_one_scatter()

      @pl.when(j + 2 < SEQ_LEN)
      def _fire_next_gather():
        pltpu.async_copy(table_sh.at[idx_v.at[j + 2]], rows_v.at[nb], sem_g)

      pltpu.make_async_copy(
          table_sh.at[idx_v.at[j]], rows_v.at[b], sem_g).wait()
      pltpu.async_copy(
          rows_v.at[b], out_hbm.at[j, pl.ds(base, S_PER_W)], sem_s)
      return carry

    lax.fori_loop(0, SEQ_LEN, body, 0)
    _drain_one_scatter()
    _drain_one_scatter()

  return gather_kernel


_gather = _make_kernel()


@jax.jit
def kernel(env_ids, table):
  out_t = _gather(env_ids.T.astype(jnp.int32), table)
  return jnp.transpose(out_t, (1, 0, 2))


# speedup vs baseline: 1.0023x; 1.0023x over previous
import functools
import jax
import jax.numpy as jnp
from jax import lax
from jax.experimental import pallas as pl
from jax.experimental.pallas import tpu as pltpu
from jax.experimental.pallas import tpu_sc as plsc

NUM_ROWS = 1000
D = 128
NSEQ = 4096
SEQ_LEN = 50
NC, NS = 2, 16
NW = NC * NS
S_PER_W = NSEQ // NW


def _make_kernel():
  mesh = plsc.VectorSubcoreMesh(core_axis_name="c", subcore_axis_name="s")

  @functools.partial(
      pl.kernel,
      out_type=jax.ShapeDtypeStruct((SEQ_LEN, NSEQ, D), jnp.float32),
      mesh=mesh,
      scratch_types=[
          pltpu.VMEM((SEQ_LEN, S_PER_W), jnp.int32),
          pltpu.VMEM((3, S_PER_W, D), jnp.float32),
          pltpu.VMEM_SHARED((NUM_ROWS, D), jnp.float32),
          pltpu.SemaphoreType.DMA,
          pltpu.SemaphoreType.DMA,
      ],
  )
  def gather_kernel(idx_hbm, table_hbm, out_hbm, idx_v, rows_v, table_sh,
                    sem_g, sem_s):
    sid = lax.axis_index("s")
    wid = sid * NC + lax.axis_index("c")
    base = wid * S_PER_W
    pltpu.sync_copy(idx_hbm.at[:, pl.ds(base, S_PER_W)], idx_v)

    @pl.when(sid == 0)
    def _stage_table():
      pltpu.sync_copy(table_hbm, table_sh)

    plsc.subcore_barrier()

    pltpu.async_copy(table_sh.at[idx_v.at[0]], rows_v.at[0], sem_g)

    def _drain_one_scatter():
      pltpu.make_async_copy(
          rows_v.at[0], out_hbm.at[0, pl.ds(base, S_PER_W)], sem_s).wait()

    def body(j, carry):
      b = lax.rem(j, 3)
      nb = lax.rem(j + 1, 3)

      @pl.when(j >= 2)
      def _wait_scatter_jm2():
        _drain_one_scatter()

      @pl.when(j + 1 < SEQ_LEN)
      def _fire_next_gather():
        pltpu.async_copy(table_sh.at[idx_v.at[j + 1]], rows_v.at[nb], sem_g)

      pltpu.make_async_copy(
          table_sh.at[idx_v.at[j]], rows_v.at[b], sem_g).wait()
      pltpu.async_copy(
          rows_v.at[b], out_hbm.at[j, pl.ds(base, S_PER_W)], sem_s)
      return carry

    lax.fori_loop(0, SEQ_LEN, body, 0)
    _drain_one_scatter()
    _drain_one_scatter()

  return gather_kernel


_gather = _make_kernel()


@jax.jit
def kernel(env_ids, table):
  out_t = _gather(env_ids.T.astype(jnp.int32), table)
  return jnp.transpose(out_t, (1, 0, 2))
